# baseline (speedup 1.0000x reference)
import functools

import jax
import jax.numpy as jnp
from jax import lax
from jax.experimental import pallas as pl
from jax.experimental.pallas import tpu as pltpu
from jax.experimental.pallas import tpu_sc as plsc

NC = 2
NS = 16
NW = NC * NS
CHUNK = 128


def _make_sc_deg(T, n_pad, rps, H):
    mesh = plsc.VectorSubcoreMesh(core_axis_name="c", subcore_axis_name="s")

    @functools.partial(
        pl.kernel,
        mesh=mesh,
        out_type=jax.ShapeDtypeStruct((NC, n_pad, H), jnp.float32),
        scratch_types=[
            pltpu.VMEM((T, CHUNK), jnp.int32),
            pltpu.VMEM((CHUNK, H), jnp.float32),
            pltpu.VMEM_SHARED((n_pad, H), jnp.float32),
            pltpu.SemaphoreType.DMA,
        ],
    )
    def deg_kernel(dst3, ones_hbm, zeros_hbm, degp, didx, ones_v, deg_sh,
                   sem):
        c = lax.axis_index("c")
        s = lax.axis_index("s")
        wid = c * NS + s
        pltpu.sync_copy(zeros_hbm, ones_v)
        for z in range(rps // CHUNK):
            pltpu.sync_copy(ones_v, deg_sh.at[pl.ds(s * rps + z * CHUNK,
                                                    CHUNK)])
        pltpu.sync_copy(dst3.at[wid], didx)
        pltpu.sync_copy(ones_hbm, ones_v)
        plsc.subcore_barrier()

        pltpu.async_copy(ones_v, deg_sh.at[didx.at[0]], sem, add=True)

        def body(t, carry):
            pltpu.async_copy(ones_v, deg_sh.at[didx.at[t]], sem, add=True)
            pltpu.make_async_copy(ones_v, deg_sh.at[didx.at[t - 1]],
                                  sem).wait()
            return carry

        lax.fori_loop(1, T, body, 0)
        pltpu.make_async_copy(ones_v, deg_sh.at[didx.at[T - 1]], sem).wait()
        plsc.subcore_barrier()
        pltpu.sync_copy(deg_sh.at[pl.ds(s * rps, rps)],
                        degp.at[c, pl.ds(s * rps, rps)])

    return deg_kernel


def _make_sc_layer(T, n_pad, rps, H):
    mesh = plsc.VectorSubcoreMesh(core_axis_name="c", subcore_axis_name="s")

    assert T % 4 == 0
    TH = T // 2

    @functools.partial(
        pl.kernel,
        mesh=mesh,
        out_type=jax.ShapeDtypeStruct((NC, n_pad, H), jnp.float32),
        scratch_types=[
            pltpu.VMEM((TH, CHUNK), jnp.int32),
            pltpu.VMEM((TH, CHUNK), jnp.int32),
            pltpu.VMEM((CHUNK, H), jnp.float32),
            pltpu.VMEM((CHUNK, H), jnp.float32),
            pltpu.VMEM_SHARED((n_pad, H), jnp.float32),
            pltpu.SemaphoreType.DMA,
            pltpu.SemaphoreType.DMA,
        ],
    )
    def layer_kernel(hs, src3, dst3, zeros_hbm, aggp, sidx, didx, msg_a,
                     msg_b, agg_sh, sem_a, sem_b):
        c = lax.axis_index("c")
        s = lax.axis_index("s")
        wid = c * NS + s
        pltpu.async_copy(src3.at[wid, pl.ds(0, TH)], sidx, sem_a)
        pltpu.async_copy(dst3.at[wid, pl.ds(0, TH)], didx, sem_b)
        pltpu.sync_copy(zeros_hbm, msg_b)
        for z in range(rps // CHUNK):
            pltpu.sync_copy(msg_b, agg_sh.at[pl.ds(s * rps + z * CHUNK,
                                                   CHUNK)])
        pltpu.make_async_copy(src3.at[wid, pl.ds(0, TH)], sidx, sem_a).wait()
        pltpu.make_async_copy(dst3.at[wid, pl.ds(0, TH)], didx, sem_b).wait()
        plsc.subcore_barrier()

        for half in range(2):
            if half:
                pltpu.sync_copy(src3.at[wid, pl.ds(half * TH, TH)], sidx)
                pltpu.sync_copy(dst3.at[wid, pl.ds(half * TH, TH)], didx)

            HB = CHUNK // 2

            def gather(t, msg, sem):
                pltpu.async_copy(hs.at[sidx.at[t, pl.ds(0, HB)]],
                                 msg.at[pl.ds(0, HB)], sem)
                pltpu.async_copy(hs.at[sidx.at[t, pl.ds(HB, HB)]],
                                 msg.at[pl.ds(HB, HB)], sem)

            def gwait(t, msg, sem):
                pltpu.make_async_copy(hs.at[sidx.at[t, pl.ds(0, HB)]],
                                      msg.at[pl.ds(0, HB)], sem).wait()
                pltpu.make_async_copy(hs.at[sidx.at[t, pl.ds(HB, HB)]],
                                      msg.at[pl.ds(HB, HB)], sem).wait()

            gather(0, msg_a, sem_a)

            def body(p, carry):
                t0 = 2 * p
                gwait(t0, msg_a, sem_a)
                gather(t0 + 1, msg_b, sem_b)
                pltpu.sync_copy(msg_a, agg_sh.at[didx.at[t0]], add=True)
                tn = jnp.minimum(t0 + 2, TH - 2)
                gather(tn, msg_a, sem_a)
                gwait(t0 + 1, msg_b, sem_b)
                pltpu.sync_copy(msg_b, agg_sh.at[didx.at[t0 + 1]], add=True)
                return carry

            lax.fori_loop(0, TH // 2, body, 0)
            gwait(TH - 2, msg_a, sem_a)

        plsc.subcore_barrier()
        pltpu.sync_copy(agg_sh.at[pl.ds(s * rps, rps)],
                        aggp.at[c, pl.ds(s * rps, rps)])

    return layer_kernel


def _tc_prep(x0, x1, degp, el, ev, n_pad, br, in_dim, H):
    grid = (n_pad // br,)

    def body(x0_ref, x1_ref, degp_ref, el_ref, ev_ref, hs_ref, isd_ref,
             sde_ref):
        io = lax.broadcasted_iota(jnp.int32, (br, in_dim), 1)
        oh0 = (x0_ref[...][:, None] == io).astype(jnp.float32)
        oh1 = (x1_ref[...][:, None] == io).astype(jnp.float32)
        h = (jnp.dot(oh0, el_ref[...], preferred_element_type=jnp.float32,
                     precision=lax.Precision.HIGHEST)
             + jnp.dot(oh1, ev_ref[...], preferred_element_type=jnp.float32,
                       precision=lax.Precision.HIGHEST))
        deg = jnp.maximum(degp_ref[0, :, 0:1] + degp_ref[1, :, 0:1], 1.0)
        isd = lax.rsqrt(deg)
        hs_ref[...] = h * isd
        isd_ref[...] = isd[:, 0]
        sde_ref[...] = jnp.sqrt(deg)[:, 0]

    return pl.pallas_call(
        body,
        grid=grid,
        in_specs=[
            pl.BlockSpec((br,), lambda i: (i,)),
            pl.BlockSpec((br,), lambda i: (i,)),
            pl.BlockSpec((NC, br, H), lambda i: (0, i, 0)),
            pl.BlockSpec((in_dim, H), lambda i: (0, 0)),
            pl.BlockSpec((in_dim, H), lambda i: (0, 0)),
        ],
        out_specs=[
            pl.BlockSpec((br, H), lambda i: (i, 0)),
            pl.BlockSpec((br,), lambda i: (i,)),
            pl.BlockSpec((br,), lambda i: (i,)),
        ],
        out_shape=[
            jax.ShapeDtypeStruct((n_pad, H), jnp.float32),
            jax.ShapeDtypeStruct((n_pad,), jnp.float32),
            jax.ShapeDtypeStruct((n_pad,), jnp.float32),
        ],
    )(x0, x1, degp, el, ev)


def _layer_math(aggp_ref, hs_ref, isd_ref, sde_ref, w_ref, b_ref, g_ref,
                bb_ref):
    isd = isd_ref[...][:, None]
    agg = (aggp_ref[0] + aggp_ref[1]) * isd
    nh = jnp.dot(agg, w_ref[...], preferred_element_type=jnp.float32,
                 precision=lax.Precision.HIGHEST) + b_ref[...]
    nh = jnp.maximum(nh, 0.0)
    h2 = hs_ref[...] * sde_ref[...][:, None] + nh
    mu = jnp.mean(h2, axis=-1, keepdims=True)
    d = h2 - mu
    var = jnp.mean(d * d, axis=-1, keepdims=True)
    hn = d * lax.rsqrt(var + 1e-5) * g_ref[...] + bb_ref[...]
    return hn, isd


def _tc_layer(aggp, hs, isd, sde, w, b, g, bb, n_pad, br, H):
    grid = (n_pad // br,)

    def body(aggp_ref, hs_ref, isd_ref, sde_ref, w_ref, b_ref, g_ref, bb_ref,
             hs2_ref):
        hn, isd_c = _layer_math(aggp_ref, hs_ref, isd_ref, sde_ref, w_ref,
                                b_ref, g_ref, bb_ref)
        hs2_ref[...] = hn * isd_c

    return pl.pallas_call(
        body,
        grid=grid,
        in_specs=[
            pl.BlockSpec((NC, br, H), lambda i: (0, i, 0)),
            pl.BlockSpec((br, H), lambda i: (i, 0)),
            pl.BlockSpec((br,), lambda i: (i,)),
            pl.BlockSpec((br,), lambda i: (i,)),
            pl.BlockSpec((H, H), lambda i: (0, 0)),
            pl.BlockSpec((H,), lambda i: (0,)),
            pl.BlockSpec((H,), lambda i: (0,)),
            pl.BlockSpec((H,), lambda i: (0,)),
        ],
        out_specs=pl.BlockSpec((br, H), lambda i: (i, 0)),
        out_shape=jax.ShapeDtypeStruct((n_pad, H), jnp.float32),
    )(aggp, hs, isd, sde, w, b, g, bb)


def _tc_final(aggp, hs, isd, sde, w, b, g, bb, wo, bo, maskf, n_pad, br, H,
              out_dim):
    grid = (n_pad // br,)

    def body(aggp_ref, hs_ref, isd_ref, sde_ref, w_ref, b_ref, g_ref, bb_ref,
             wo_ref, bo_ref, m_ref, out_ref):
        hn, _ = _layer_math(aggp_ref, hs_ref, isd_ref, sde_ref, w_ref, b_ref,
                            g_ref, bb_ref)
        out = jnp.dot(hn, wo_ref[...], preferred_element_type=jnp.float32,
                      precision=lax.Precision.HIGHEST) + bo_ref[...]
        out_ref[...] = out * m_ref[...][:, None]

    return pl.pallas_call(
        body,
        grid=grid,
        in_specs=[
            pl.BlockSpec((NC, br, H), lambda i: (0, i, 0)),
            pl.BlockSpec((br, H), lambda i: (i, 0)),
            pl.BlockSpec((br,), lambda i: (i,)),
            pl.BlockSpec((br,), lambda i: (i,)),
            pl.BlockSpec((H, H), lambda i: (0, 0)),
            pl.BlockSpec((H,), lambda i: (0,)),
            pl.BlockSpec((H,), lambda i: (0,)),
            pl.BlockSpec((H,), lambda i: (0,)),
            pl.BlockSpec((H, out_dim), lambda i: (0, 0)),
            pl.BlockSpec((out_dim,), lambda i: (0,)),
            pl.BlockSpec((br,), lambda i: (i,)),
        ],
        out_specs=pl.BlockSpec((br, out_dim), lambda i: (i, 0)),
        out_shape=jax.ShapeDtypeStruct((n_pad, out_dim), jnp.float32),
    )(aggp, hs, isd, sde, w, b, g, bb, wo, bo, maskf)


def kernel(x, edge_index, root_mask, embed_label, embed_value, W, b, ln_scale,
           ln_bias, W_out, b_out):
    N = x.shape[0]
    E = edge_index.shape[1]
    in_dim, H = embed_label.shape
    L = W.shape[0]
    out_dim = W_out.shape[1]

    rps = -(-N // (NS * 64)) * 64
    n_pad = rps * NS
    if n_pad == N:
        rps += 64
        n_pad = rps * NS
    T = -(-E // (NW * CHUNK))
    T = -(-T // 4) * 4
    e_pad = NW * T * CHUNK
    br = 1024 if n_pad % 1024 == 0 else rps
    assert n_pad % br == 0

    src = edge_index[0].astype(jnp.int32)
    dst = edge_index[1].astype(jnp.int32)
    pid = jnp.arange(e_pad - E, dtype=jnp.int32)
    src3 = jnp.concatenate([src, pid % N]).reshape(NW, T, CHUNK)
    dst3 = jnp.concatenate([dst, N + pid % (n_pad - N)]).reshape(NW, T, CHUNK)

    assert rps % CHUNK == 0
    onesH = jnp.ones((CHUNK, H), jnp.float32)
    zerosH = jnp.zeros((CHUNK, H), jnp.float32)
    x_pad = jnp.pad(x.astype(jnp.int32), ((0, n_pad - N), (0, 0)))
    maskf = jnp.pad(root_mask.astype(jnp.float32), (0, n_pad - N))

    sc_deg = _make_sc_deg(T, n_pad, rps, H)
    sc_layer = _make_sc_layer(T, n_pad, rps, H)

    degp = sc_deg(dst3, onesH, zerosH)
    hs, isd, sde = _tc_prep(x_pad[:, 0], x_pad[:, 1], degp, embed_label,
                            embed_value, n_pad, br, in_dim, H)
    for i in range(L - 1):
        aggp = sc_layer(hs, src3, dst3, zerosH)
        hs = _tc_layer(aggp, hs, isd, sde, W[i], b[i], ln_scale[i],
                       ln_bias[i], n_pad, br, H)
    aggp = sc_layer(hs, src3, dst3, zerosH)
    out = _tc_final(aggp, hs, isd, sde, W[L - 1], b[L - 1], ln_scale[L - 1],
                    ln_bias[L - 1], W_out, b_out, maskf, n_pad, br, H,
                    out_dim)
    return out[:N]


# speedup vs baseline: 1.0012x; 1.0012x over previous
import functools

import jax
import jax.numpy as jnp
from jax import lax
from jax.experimental import pallas as pl
from jax.experimental.pallas import tpu as pltpu
from jax.experimental.pallas import tpu_sc as plsc

NC = 2
NS = 16
NW = NC * NS
CHUNK = 128


def _make_sc_deg(T, n_pad, rps, H):
    mesh = plsc.VectorSubcoreMesh(core_axis_name="c", subcore_axis_name="s")

    @functools.partial(
        pl.kernel,
        mesh=mesh,
        out_type=jax.ShapeDtypeStruct((NC, n_pad, H), jnp.float32),
        scratch_types=[
            pltpu.VMEM((T, CHUNK), jnp.int32),
            pltpu.VMEM((CHUNK, H), jnp.float32),
            pltpu.VMEM_SHARED((n_pad, H), jnp.float32),
            pltpu.SemaphoreType.DMA,
        ],
    )
    def deg_kernel(dst3, ones_hbm, zeros_hbm, degp, didx, ones_v, deg_sh,
                   sem):
        c = lax.axis_index("c")
        s = lax.axis_index("s")
        wid = c * NS + s
        pltpu.sync_copy(zeros_hbm, ones_v)
        for z in range(rps // CHUNK):
            pltpu.sync_copy(ones_v, deg_sh.at[pl.ds(s * rps + z * CHUNK,
                                                    CHUNK)])
        pltpu.sync_copy(dst3.at[wid], didx)
        pltpu.sync_copy(ones_hbm, ones_v)
        plsc.subcore_barrier()

        pltpu.async_copy(ones_v, deg_sh.at[didx.at[0]], sem, add=True)

        def body(t, carry):
            pltpu.async_copy(ones_v, deg_sh.at[didx.at[t]], sem, add=True)
            pltpu.make_async_copy(ones_v, deg_sh.at[didx.at[t - 1]],
                                  sem).wait()
            return carry

        lax.fori_loop(1, T, body, 0)
        pltpu.make_async_copy(ones_v, deg_sh.at[didx.at[T - 1]], sem).wait()
        plsc.subcore_barrier()
        pltpu.sync_copy(deg_sh.at[pl.ds(s * rps, rps)],
                        degp.at[c, pl.ds(s * rps, rps)])

    return deg_kernel


def _make_sc_layer(T, n_pad, rps, H):
    mesh = plsc.VectorSubcoreMesh(core_axis_name="c", subcore_axis_name="s")

    assert T % 4 == 0
    TH = T // 2

    @functools.partial(
        pl.kernel,
        mesh=mesh,
        out_type=jax.ShapeDtypeStruct((NC, n_pad, H), jnp.float32),
        scratch_types=[
            pltpu.VMEM((TH, CHUNK), jnp.int32),
            pltpu.VMEM((TH, CHUNK), jnp.int32),
            pltpu.VMEM((CHUNK, H), jnp.float32),
            pltpu.VMEM((CHUNK, H), jnp.float32),
            pltpu.VMEM_SHARED((n_pad, H), jnp.float32),
            pltpu.SemaphoreType.DMA,
            pltpu.SemaphoreType.DMA,
        ],
    )
    def layer_kernel(hs, src3, dst3, zeros_hbm, aggp, sidx, didx, msg_a,
                     msg_b, agg_sh, sem_a, sem_b):
        c = lax.axis_index("c")
        s = lax.axis_index("s")
        wid = c * NS + s
        pltpu.async_copy(src3.at[wid, pl.ds(0, TH)], sidx, sem_a)
        pltpu.async_copy(dst3.at[wid, pl.ds(0, TH)], didx, sem_b)
        pltpu.sync_copy(zeros_hbm, msg_b)
        for z in range(rps // CHUNK):
            pltpu.sync_copy(msg_b, agg_sh.at[pl.ds(s * rps + z * CHUNK,
                                                   CHUNK)])
        pltpu.make_async_copy(src3.at[wid, pl.ds(0, TH)], sidx, sem_a).wait()
        pltpu.make_async_copy(dst3.at[wid, pl.ds(0, TH)], didx, sem_b).wait()
        plsc.subcore_barrier()

        for half in range(2):
            if half:
                pltpu.sync_copy(src3.at[wid, pl.ds(half * TH, TH)], sidx)
                pltpu.sync_copy(dst3.at[wid, pl.ds(half * TH, TH)], didx)

            HB = CHUNK // 2

            def gather(t, msg, sem):
                pltpu.async_copy(hs.at[sidx.at[t, pl.ds(0, HB)]],
                                 msg.at[pl.ds(0, HB)], sem)
                pltpu.async_copy(hs.at[sidx.at[t, pl.ds(HB, HB)]],
                                 msg.at[pl.ds(HB, HB)], sem)

            def gwait(t, msg, sem):
                pltpu.make_async_copy(hs.at[sidx.at[t, pl.ds(0, HB)]],
                                      msg.at[pl.ds(0, HB)], sem).wait()
                pltpu.make_async_copy(hs.at[sidx.at[t, pl.ds(HB, HB)]],
                                      msg.at[pl.ds(HB, HB)], sem).wait()

            gather(0, msg_a, sem_a)

            def body(p, carry):
                t0 = 2 * p
                gwait(t0, msg_a, sem_a)
                gather(t0 + 1, msg_b, sem_b)
                pltpu.sync_copy(msg_a, agg_sh.at[didx.at[t0]], add=True)
                tn = jnp.minimum(t0 + 2, TH - 2)
                gather(tn, msg_a, sem_a)
                gwait(t0 + 1, msg_b, sem_b)
                pltpu.sync_copy(msg_b, agg_sh.at[didx.at[t0 + 1]], add=True)
                return carry

            lax.fori_loop(0, TH // 2, body, 0)
            gwait(TH - 2, msg_a, sem_a)

        plsc.subcore_barrier()
        pltpu.sync_copy(agg_sh.at[pl.ds(s * rps, rps)],
                        aggp.at[c, pl.ds(s * rps, rps)])

    return layer_kernel


def _tc_prep(x0, x1, degp, el, ev, n_pad, br, in_dim, H):
    grid = (n_pad // br,)

    def body(x0_ref, x1_ref, degp_ref, el_ref, ev_ref, hs_ref, isd_ref,
             sde_ref):
        io = lax.broadcasted_iota(jnp.int32, (br, in_dim), 1)
        oh0 = (x0_ref[...][:, None] == io).astype(jnp.float32)
        oh1 = (x1_ref[...][:, None] == io).astype(jnp.float32)
        h = (jnp.dot(oh0, el_ref[...], preferred_element_type=jnp.float32,
                     precision=lax.Precision.HIGHEST)
             + jnp.dot(oh1, ev_ref[...], preferred_element_type=jnp.float32,
                       precision=lax.Precision.HIGHEST))
        deg = jnp.maximum(degp_ref[0, :, 0:1] + degp_ref[1, :, 0:1], 1.0)
        isd = lax.rsqrt(deg)
        hs_ref[...] = h * isd
        isd_ref[...] = isd[:, 0]
        sde_ref[...] = jnp.sqrt(deg)[:, 0]

    return pl.pallas_call(
        body,
        grid=grid,
        in_specs=[
            pl.BlockSpec((br,), lambda i: (i,)),
            pl.BlockSpec((br,), lambda i: (i,)),
            pl.BlockSpec((NC, br, H), lambda i: (0, i, 0)),
            pl.BlockSpec((in_dim, H), lambda i: (0, 0)),
            pl.BlockSpec((in_dim, H), lambda i: (0, 0)),
        ],
        out_specs=[
            pl.BlockSpec((br, H), lambda i: (i, 0)),
            pl.BlockSpec((br,), lambda i: (i,)),
            pl.BlockSpec((br,), lambda i: (i,)),
        ],
        out_shape=[
            jax.ShapeDtypeStruct((n_pad, H), jnp.float32),
            jax.ShapeDtypeStruct((n_pad,), jnp.float32),
            jax.ShapeDtypeStruct((n_pad,), jnp.float32),
        ],
    )(x0, x1, degp, el, ev)


def _layer_math(aggp_ref, hs_ref, isd_ref, sde_ref, w_ref, b_ref, g_ref,
                bb_ref):
    isd = isd_ref[...][:, None]
    agg = (aggp_ref[0] + aggp_ref[1]) * isd
    nh = jnp.dot(agg, w_ref[...], preferred_element_type=jnp.float32,
                 precision=lax.Precision.HIGHEST) + b_ref[...]
    nh = jnp.maximum(nh, 0.0)
    h2 = hs_ref[...] * sde_ref[...][:, None] + nh
    mu = jnp.mean(h2, axis=-1, keepdims=True)
    d = h2 - mu
    var = jnp.mean(d * d, axis=-1, keepdims=True)
    hn = d * lax.rsqrt(var + 1e-5) * g_ref[...] + bb_ref[...]
    return hn, isd


def _tc_layer(aggp, hs, isd, sde, w, b, g, bb, n_pad, br, H):
    grid = (n_pad // br,)

    def body(aggp_ref, hs_ref, isd_ref, sde_ref, w_ref, b_ref, g_ref, bb_ref,
             hs2_ref):
        hn, isd_c = _layer_math(aggp_ref, hs_ref, isd_ref, sde_ref, w_ref,
                                b_ref, g_ref, bb_ref)
        hs2_ref[...] = hn * isd_c

    return pl.pallas_call(
        body,
        grid=grid,
        in_specs=[
            pl.BlockSpec((NC, br, H), lambda i: (0, i, 0)),
            pl.BlockSpec((br, H), lambda i: (i, 0)),
            pl.BlockSpec((br,), lambda i: (i,)),
            pl.BlockSpec((br,), lambda i: (i,)),
            pl.BlockSpec((H, H), lambda i: (0, 0)),
            pl.BlockSpec((H,), lambda i: (0,)),
            pl.BlockSpec((H,), lambda i: (0,)),
            pl.BlockSpec((H,), lambda i: (0,)),
        ],
        out_specs=pl.BlockSpec((br, H), lambda i: (i, 0)),
        out_shape=jax.ShapeDtypeStruct((n_pad, H), jnp.float32),
    )(aggp, hs, isd, sde, w, b, g, bb)


def _tc_final(aggp, hs, isd, sde, w, b, g, bb, wo, bo, maskf, n_pad, br, H,
              out_dim):
    grid = (n_pad // br,)

    def body(aggp_ref, hs_ref, isd_ref, sde_ref, w_ref, b_ref, g_ref, bb_ref,
             wo_ref, bo_ref, m_ref, out_ref):
        hn, _ = _layer_math(aggp_ref, hs_ref, isd_ref, sde_ref, w_ref, b_ref,
                            g_ref, bb_ref)
        out = jnp.dot(hn, wo_ref[...], preferred_element_type=jnp.float32,
                      precision=lax.Precision.HIGHEST) + bo_ref[...]
        out_ref[...] = out * m_ref[...][:, None]

    return pl.pallas_call(
        body,
        grid=grid,
        in_specs=[
            pl.BlockSpec((NC, br, H), lambda i: (0, i, 0)),
            pl.BlockSpec((br, H), lambda i: (i, 0)),
            pl.BlockSpec((br,), lambda i: (i,)),
            pl.BlockSpec((br,), lambda i: (i,)),
            pl.BlockSpec((H, H), lambda i: (0, 0)),
            pl.BlockSpec((H,), lambda i: (0,)),
            pl.BlockSpec((H,), lambda i: (0,)),
            pl.BlockSpec((H,), lambda i: (0,)),
            pl.BlockSpec((H, out_dim), lambda i: (0, 0)),
            pl.BlockSpec((out_dim,), lambda i: (0,)),
            pl.BlockSpec((br,), lambda i: (i,)),
        ],
        out_specs=pl.BlockSpec((br, out_dim), lambda i: (i, 0)),
        out_shape=jax.ShapeDtypeStruct((n_pad, out_dim), jnp.float32),
    )(aggp, hs, isd, sde, w, b, g, bb, wo, bo, maskf)


def kernel(x, edge_index, root_mask, embed_label, embed_value, W, b, ln_scale,
           ln_bias, W_out, b_out):
    N = x.shape[0]
    E = edge_index.shape[1]
    in_dim, H = embed_label.shape
    L = W.shape[0]
    out_dim = W_out.shape[1]

    rps = -(-N // (NS * 64)) * 64
    n_pad = rps * NS
    if n_pad == N:
        rps += 64
        n_pad = rps * NS
    T = -(-E // (NW * CHUNK))
    T = -(-T // 4) * 4
    e_pad = NW * T * CHUNK
    br = 1024 if n_pad % 1024 == 0 else rps
    assert n_pad % br == 0

    src = edge_index[0].astype(jnp.int32)
    dst = edge_index[1].astype(jnp.int32)
    pid = jnp.arange(e_pad - E, dtype=jnp.int32)
    pad_src = jnp.minimum(pid & 8191, N - 1)
    pad_dst = N + (pid & 127)
    assert n_pad - N >= 128
    src3 = jnp.concatenate([src, pad_src]).reshape(NW, T, CHUNK)
    dst3 = jnp.concatenate([dst, pad_dst]).reshape(NW, T, CHUNK)

    assert rps % CHUNK == 0
    onesH = jnp.ones((CHUNK, H), jnp.float32)
    zerosH = jnp.zeros((CHUNK, H), jnp.float32)
    x_pad = jnp.pad(x.astype(jnp.int32), ((0, n_pad - N), (0, 0)))
    maskf = jnp.pad(root_mask.astype(jnp.float32), (0, n_pad - N))

    sc_deg = _make_sc_deg(T, n_pad, rps, H)
    sc_layer = _make_sc_layer(T, n_pad, rps, H)

    degp = sc_deg(dst3, onesH, zerosH)
    hs, isd, sde = _tc_prep(x_pad[:, 0], x_pad[:, 1], degp, embed_label,
                            embed_value, n_pad, br, in_dim, H)
    for i in range(L - 1):
        aggp = sc_layer(hs, src3, dst3, zerosH)
        hs = _tc_layer(aggp, hs, isd, sde, W[i], b[i], ln_scale[i],
                       ln_bias[i], n_pad, br, H)
    aggp = sc_layer(hs, src3, dst3, zerosH)
    out = _tc_final(aggp, hs, isd, sde, W[L - 1], b[L - 1], ln_scale[L - 1],
                    ln_bias[L - 1], W_out, b_out, maskf, n_pad, br, H,
                    out_dim)
    return out[:N]
